# baseline (speedup 1.0000x reference)
import functools

import jax
import jax.numpy as jnp
from jax import lax
from jax.experimental import pallas as pl
from jax.experimental.pallas import tpu as pltpu
from jax.experimental.pallas import tpu_sc as plsc

B = 4096
NA = 10
NE = 20
NC = 2
NS = 16
NW = NC * NS
BPW = B // NW
MOV_W = BPW * NE * 2
GOAL_W = BPW * NA * 3
N_PHYS = BPW * NA
N_MOVE = BPW * NE


def _vsqrt(x):
    i = plsc.bitcast(x, jnp.int32)
    i = jnp.int32(0x5F3759DF) - (i >> 1)
    y = plsc.bitcast(i, jnp.float32)
    for _ in range(3):
        y = y * (1.5 - 0.5 * x * y * y)
    return x * y


def _sc_body(mov_hbm, loc_hbm, goal_hbm, out_hbm, mov_v, loc_v, goal_v, res_v):
    wid = lax.axis_index("s") * NC + lax.axis_index("c")
    pltpu.sync_copy(mov_hbm.at[pl.ds(wid * MOV_W, MOV_W)], mov_v)
    pltpu.sync_copy(loc_hbm.at[pl.ds(wid * MOV_W, MOV_W)], loc_v)
    pltpu.sync_copy(goal_hbm.at[pl.ds(wid * GOAL_W, GOAL_W)], goal_v)

    lane = lax.broadcasted_iota(jnp.int32, (16,), 0)
    zero = jnp.zeros((16,), jnp.float32)

    def move_step(i, acc):
        idx = 32 * i + 2 * lane
        mx = plsc.load_gather(mov_v, [idx])
        my = plsc.load_gather(mov_v, [idx + 1])
        return acc + _vsqrt(mx * mx + my * my)

    acc_m = lax.fori_loop(0, N_MOVE // 16, move_step, zero)

    def phys_step(t, acc):
        s = 16 * t + lane
        si = 3 * s
        gx = plsc.load_gather(goal_v, [si])
        gy = plsc.load_gather(goal_v, [si + 1])
        gid = plsc.load_gather(goal_v, [si + 2]).astype(jnp.int32)
        q = (s // NA) * (NE * 2) + 2 * gid
        lx = plsc.load_gather(loc_v, [q]) + plsc.load_gather(mov_v, [q])
        ly = plsc.load_gather(loc_v, [q + 1]) + plsc.load_gather(mov_v, [q + 1])
        dx = lx - gx
        dy = ly - gy
        return acc + _vsqrt(dx * dx + dy * dy)

    acc_p = lax.fori_loop(0, N_PHYS // 16, phys_step, zero)

    res_v[...] = acc_m + 2.0 * acc_p
    pltpu.sync_copy(res_v, out_hbm.at[wid])


@jax.jit
def _sc_cost(mov1d, loc1d, goal1d):
    mesh = plsc.VectorSubcoreMesh(core_axis_name="c", subcore_axis_name="s")
    f = pl.kernel(
        _sc_body,
        out_type=jax.ShapeDtypeStruct((NW, 16), jnp.float32),
        mesh=mesh,
        scratch_types=[
            pltpu.VMEM((MOV_W,), jnp.float32),
            pltpu.VMEM((MOV_W,), jnp.float32),
            pltpu.VMEM((GOAL_W,), jnp.float32),
            pltpu.VMEM((16,), jnp.float32),
        ],
        compiler_params=pltpu.CompilerParams(needs_layout_passes=False),
    )
    return f(mov1d, loc1d, goal1d)


def kernel(movements, goal_predictions, utterances, locations, goals):
    partials = _sc_cost(
        movements.reshape(-1), locations.reshape(-1), goals.reshape(-1)
    )
    return jnp.sum(partials)


# speedup vs baseline: 5.7080x; 5.7080x over previous
import jax
import jax.numpy as jnp
from jax import lax
from jax.experimental import pallas as pl
from jax.experimental.pallas import tpu as pltpu
from jax.experimental.pallas import tpu_sc as plsc

B = 4096
NA = 10
NE = 20
NC = 2
NS = 16
NW = NC * NS
NT = B // 128
KS = 128 // 16


def _vsqrt(x):
    i = plsc.bitcast(x, jnp.int32)
    i = jnp.int32(0x5F3759DF) - (i >> 1)
    y = plsc.bitcast(i, jnp.float32)
    for _ in range(3):
        y = y * (1.5 - 0.5 * x * y * y)
    return x * y


def _sc_body(mov_hbm, loc_hbm, goal_hbm, out_hbm, mov_v, loc_v, goal_v, res_v, sem):
    wid = lax.axis_index("s") * NC + lax.axis_index("c")
    copies = []
    for e in range(NE):
        copies.append(
            pltpu.async_copy(mov_hbm.at[e, wid], mov_v.at[pl.ds(2 * e, 2)], sem)
        )
        copies.append(
            pltpu.async_copy(loc_hbm.at[e, wid], loc_v.at[pl.ds(2 * e, 2)], sem)
        )
    for j in range(NA):
        copies.append(
            pltpu.async_copy(goal_hbm.at[j, :, wid], goal_v.at[pl.ds(3 * j, 3)], sem)
        )
    for c in copies:
        c.wait()

    lane = lax.broadcasted_iota(jnp.int32, (16,), 0)

    def row(v):
        return jnp.full((16,), v, jnp.int32)

    def k_step(k, acc):
        col = k * 16 + lane
        am = jnp.zeros((16,), jnp.float32)
        for e in range(NE):
            mx = plsc.load_gather(mov_v, [row(2 * e), col])
            my = plsc.load_gather(mov_v, [row(2 * e + 1), col])
            am = am + _vsqrt(mx * mx + my * my)
        ap = jnp.zeros((16,), jnp.float32)
        for j in range(NA):
            gx = plsc.load_gather(goal_v, [row(3 * j), col])
            gy = plsc.load_gather(goal_v, [row(3 * j + 1), col])
            g = plsc.load_gather(goal_v, [row(3 * j + 2), col]).astype(jnp.int32)
            g2 = g * 2
            lx = plsc.load_gather(loc_v, [g2, col]) + plsc.load_gather(
                mov_v, [g2, col]
            )
            ly = plsc.load_gather(loc_v, [g2 + 1, col]) + plsc.load_gather(
                mov_v, [g2 + 1, col]
            )
            dx = lx - gx
            dy = ly - gy
            ap = ap + _vsqrt(dx * dx + dy * dy)
        return acc + am + 2.0 * ap

    res_v[...] = lax.fori_loop(0, KS, k_step, jnp.zeros((16,), jnp.float32))
    pltpu.sync_copy(res_v, out_hbm.at[wid])


@jax.jit
def _sc_cost(mov4, loc4, goal4):
    mesh = plsc.VectorSubcoreMesh(core_axis_name="c", subcore_axis_name="s")
    f = pl.kernel(
        _sc_body,
        out_type=jax.ShapeDtypeStruct((NW, 16), jnp.float32),
        mesh=mesh,
        scratch_types=[
            pltpu.VMEM((NE * 2, 128), jnp.float32),
            pltpu.VMEM((NE * 2, 128), jnp.float32),
            pltpu.VMEM((NA * 3, 128), jnp.float32),
            pltpu.VMEM((16,), jnp.float32),
            pltpu.SemaphoreType.DMA,
        ],
        compiler_params=pltpu.CompilerParams(needs_layout_passes=False),
    )
    return jnp.sum(f(mov4, loc4, goal4))


def kernel(movements, goal_predictions, utterances, locations, goals):
    mov4 = movements.transpose(1, 2, 0).reshape(NE, 2, NT, 128).transpose(0, 2, 1, 3)
    loc4 = locations.transpose(1, 2, 0).reshape(NE, 2, NT, 128).transpose(0, 2, 1, 3)
    goal4 = goals.transpose(1, 2, 0).reshape(NA, 3, NT, 128)
    return _sc_cost(mov4, loc4, goal4)


# speedup vs baseline: 5.8254x; 1.0206x over previous
import jax
import jax.numpy as jnp
from jax import lax
from jax.experimental import pallas as pl
from jax.experimental.pallas import tpu as pltpu
from jax.experimental.pallas import tpu_sc as plsc

B = 4096
NA = 10
NE = 20
NC = 2
NS = 16
NW = NC * NS
NT = B // 128
KS = 128 // 16


def _vsqrt(x):
    i = plsc.bitcast(x, jnp.int32)
    i = jnp.int32(0x5F3759DF) - (i >> 1)
    y = plsc.bitcast(i, jnp.float32)
    for _ in range(2):
        y = y * (1.5 - 0.5 * x * y * y)
    return x * y


def _sc_body(
    mov_hbm, loc_hbm, goal_hbm, out_hbm, mov_v, loc_v, goal_v, res_v, sem_m, sem_lg
):
    wid = lax.axis_index("s") * NC + lax.axis_index("c")
    cp_m = pltpu.async_copy(mov_hbm.at[:, wid], mov_v, sem_m)
    cp_l = pltpu.async_copy(loc_hbm.at[:, wid], loc_v, sem_lg)
    cp_g = pltpu.async_copy(goal_hbm.at[:, wid], goal_v, sem_lg)

    lane = lax.broadcasted_iota(jnp.int32, (16,), 0)

    def row(v):
        return jnp.full((16,), v, jnp.int32)

    cp_m.wait()

    def move_step(k, acc):
        col = k * 16 + lane
        am = jnp.zeros((16,), jnp.float32)
        for e in range(NE):
            mx = plsc.load_gather(mov_v, [row(e), col])
            my = plsc.load_gather(mov_v, [row(e), col + 128])
            am = am + _vsqrt(mx * mx + my * my)
        return acc + am

    acc = lax.fori_loop(0, KS, move_step, jnp.zeros((16,), jnp.float32))

    cp_l.wait()
    cp_g.wait()

    def phys_step(k, acc):
        col = k * 16 + lane
        ap = jnp.zeros((16,), jnp.float32)
        for j in range(NA):
            gx = plsc.load_gather(goal_v, [row(j), col])
            gy = plsc.load_gather(goal_v, [row(j), col + 128])
            g = plsc.load_gather(goal_v, [row(j), col + 256]).astype(jnp.int32)
            lx = plsc.load_gather(loc_v, [g, col]) + plsc.load_gather(
                mov_v, [g, col]
            )
            ly = plsc.load_gather(loc_v, [g, col + 128]) + plsc.load_gather(
                mov_v, [g, col + 128]
            )
            dx = lx - gx
            dy = ly - gy
            ap = ap + _vsqrt(dx * dx + dy * dy)
        return acc + 2.0 * ap

    res_v[...] = lax.fori_loop(0, KS, phys_step, acc)
    pltpu.sync_copy(res_v, out_hbm.at[wid])


@jax.jit
def _sc_cost(mov3, loc3, goal3):
    mesh = plsc.VectorSubcoreMesh(core_axis_name="c", subcore_axis_name="s")
    f = pl.kernel(
        _sc_body,
        out_type=jax.ShapeDtypeStruct((NW, 16), jnp.float32),
        mesh=mesh,
        scratch_types=[
            pltpu.VMEM((NE, 256), jnp.float32),
            pltpu.VMEM((NE, 256), jnp.float32),
            pltpu.VMEM((NA, 384), jnp.float32),
            pltpu.VMEM((16,), jnp.float32),
            pltpu.SemaphoreType.DMA,
            pltpu.SemaphoreType.DMA,
        ],
        compiler_params=pltpu.CompilerParams(needs_layout_passes=False),
    )
    return jnp.sum(f(mov3, loc3, goal3))


def kernel(movements, goal_predictions, utterances, locations, goals):
    mov3 = (
        movements.transpose(1, 2, 0)
        .reshape(NE, 2, NT, 128)
        .transpose(0, 2, 1, 3)
        .reshape(NE, NT, 256)
    )
    loc3 = (
        locations.transpose(1, 2, 0)
        .reshape(NE, 2, NT, 128)
        .transpose(0, 2, 1, 3)
        .reshape(NE, NT, 256)
    )
    goal3 = (
        goals.transpose(1, 2, 0)
        .reshape(NA, 3, NT, 128)
        .transpose(0, 2, 1, 3)
        .reshape(NA, NT, 384)
    )
    return _sc_cost(mov3, loc3, goal3)


# speedup vs baseline: 6.2757x; 1.0773x over previous
import jax
import jax.numpy as jnp
from jax import lax
from jax.experimental import pallas as pl
from jax.experimental.pallas import tpu as pltpu
from jax.experimental.pallas import tpu_sc as plsc

B = 4096
NA = 10
NE = 20
NC = 2
NS = 16
NW = NC * NS
NT = B // 128
KS = 128 // 16


def _vsqrt(x):
    i = plsc.bitcast(x, jnp.int32)
    i = jnp.int32(0x5F3759DF) - (i >> 1)
    y = plsc.bitcast(i, jnp.float32)
    for _ in range(2):
        y = y * (1.5 - 0.5 * x * y * y)
    return x * y


def _sc_body(
    mov_hbm, loc_hbm, goal_hbm, out_hbm, mov_v, loc_v, goal_v, res_v, sem_m, sem_lg
):
    wid = lax.axis_index("s") * NC + lax.axis_index("c")
    cps = [
        pltpu.async_copy(mov_hbm.at[:, wid, c], mov_v.at[pl.ds(c * NE, NE)], sem_m)
        for c in range(2)
    ]
    cps_lg = [
        pltpu.async_copy(loc_hbm.at[:, wid, c], loc_v.at[pl.ds(c * NE, NE)], sem_lg)
        for c in range(2)
    ] + [
        pltpu.async_copy(goal_hbm.at[c, :, wid], goal_v.at[pl.ds(c * NA, NA)], sem_lg)
        for c in range(3)
    ]

    lane = lax.broadcasted_iota(jnp.int32, (16,), 0)

    def row(v):
        return jnp.full((16,), v, jnp.int32)

    for c in cps:
        c.wait()

    def move_step(k, acc):
        col = k * 16 + lane
        am = jnp.zeros((16,), jnp.float32)
        for e in range(NE):
            mx = plsc.load_gather(mov_v, [row(e), col])
            my = plsc.load_gather(mov_v, [row(NE + e), col])
            am = am + _vsqrt(mx * mx + my * my)
        return acc + am

    acc = lax.fori_loop(0, KS, move_step, jnp.zeros((16,), jnp.float32))

    for c in cps_lg:
        c.wait()

    def phys_step(k, acc):
        col = k * 16 + lane
        ap = jnp.zeros((16,), jnp.float32)
        for j in range(NA):
            gx = plsc.load_gather(goal_v, [row(j), col])
            gy = plsc.load_gather(goal_v, [row(NA + j), col])
            g = plsc.load_gather(goal_v, [row(2 * NA + j), col]).astype(jnp.int32)
            lx = plsc.load_gather(loc_v, [g, col]) + plsc.load_gather(
                mov_v, [g, col]
            )
            ly = plsc.load_gather(loc_v, [g + NE, col]) + plsc.load_gather(
                mov_v, [g + NE, col]
            )
            dx = lx - gx
            dy = ly - gy
            ap = ap + _vsqrt(dx * dx + dy * dy)
        return acc + 2.0 * ap

    res_v[...] = lax.fori_loop(0, KS, phys_step, acc)
    pltpu.sync_copy(res_v, out_hbm.at[wid])


@jax.jit
def _sc_cost(mov4, loc4, goal4):
    mesh = plsc.VectorSubcoreMesh(core_axis_name="c", subcore_axis_name="s")
    f = pl.kernel(
        _sc_body,
        out_type=jax.ShapeDtypeStruct((NW, 16), jnp.float32),
        mesh=mesh,
        scratch_types=[
            pltpu.VMEM((2 * NE, 128), jnp.float32),
            pltpu.VMEM((2 * NE, 128), jnp.float32),
            pltpu.VMEM((3 * NA, 128), jnp.float32),
            pltpu.VMEM((16,), jnp.float32),
            pltpu.SemaphoreType.DMA,
            pltpu.SemaphoreType.DMA,
        ],
        compiler_params=pltpu.CompilerParams(needs_layout_passes=False),
    )
    return jnp.sum(f(mov4, loc4, goal4))


def kernel(movements, goal_predictions, utterances, locations, goals):
    mov4 = movements.transpose(1, 2, 0).reshape(NE, 2, NT, 128).transpose(0, 2, 1, 3)
    loc4 = locations.transpose(1, 2, 0).reshape(NE, 2, NT, 128).transpose(0, 2, 1, 3)
    goal4 = goals.transpose(2, 1, 0).reshape(3, NA, NT, 128)
    return _sc_cost(mov4, loc4, goal4)


# speedup vs baseline: 6.6769x; 1.0639x over previous
import jax
import jax.numpy as jnp
from jax import lax
from jax.experimental import pallas as pl
from jax.experimental.pallas import tpu as pltpu
from jax.experimental.pallas import tpu_sc as plsc

B = 4096
NA = 10
NE = 20
NC = 2
NS = 16
NW = NC * NS
NT = B // 128
KS = 128 // 16


def _vsqrt(x):
    i = plsc.bitcast(x, jnp.int32)
    i = jnp.int32(0x5F3759DF) - (i >> 1)
    y = plsc.bitcast(i, jnp.float32)
    for _ in range(2):
        y = y * (1.5 - 0.5 * x * y * y)
    return x * y


def _sc_body(
    mov_hbm, loc_hbm, goal_hbm, out_hbm, mov_v, loc_v, goal_v, res_v, sem_m, sem_lg
):
    wid = lax.axis_index("s") * NC + lax.axis_index("c")
    cps = [
        pltpu.async_copy(mov_hbm.at[:, wid, c], mov_v.at[pl.ds(c * NE, NE)], sem_m)
        for c in range(2)
    ]
    cps_lg = [
        pltpu.async_copy(loc_hbm.at[:, wid, c], loc_v.at[pl.ds(c * NE, NE)], sem_lg)
        for c in range(2)
    ] + [
        pltpu.async_copy(goal_hbm.at[c, :, wid], goal_v.at[pl.ds(c * NA, NA)], sem_lg)
        for c in range(3)
    ]

    lane = lax.broadcasted_iota(jnp.int32, (16,), 0)

    def row(v):
        return jnp.full((16,), v, jnp.int32)

    for c in cps:
        c.wait()

    def move_step(t, acc):
        e = t // KS
        col = (t % KS) * 16 + lane
        mx = plsc.load_gather(mov_v, [e + lane * 0, col])
        my = plsc.load_gather(mov_v, [e + NE + lane * 0, col])
        return acc + _vsqrt(mx * mx + my * my)

    acc = lax.fori_loop(0, KS * NE, move_step, jnp.zeros((16,), jnp.float32))

    for c in cps_lg:
        c.wait()

    def phys_step(t, acc):
        j = t // KS
        col = (t % KS) * 16 + lane
        gx = plsc.load_gather(goal_v, [j + lane * 0, col])
        gy = plsc.load_gather(goal_v, [j + NA + lane * 0, col])
        g = plsc.load_gather(goal_v, [j + 2 * NA + lane * 0, col]).astype(jnp.int32)
        lx = plsc.load_gather(loc_v, [g, col]) + plsc.load_gather(mov_v, [g, col])
        ly = plsc.load_gather(loc_v, [g + NE, col]) + plsc.load_gather(
            mov_v, [g + NE, col]
        )
        dx = lx - gx
        dy = ly - gy
        return acc + 2.0 * _vsqrt(dx * dx + dy * dy)

    res_v[...] = lax.fori_loop(0, KS * NA, phys_step, acc)
    pltpu.sync_copy(res_v, out_hbm.at[wid])


@jax.jit
def _sc_cost(mov4, loc4, goal4):
    mesh = plsc.VectorSubcoreMesh(core_axis_name="c", subcore_axis_name="s")
    f = pl.kernel(
        _sc_body,
        out_type=jax.ShapeDtypeStruct((NW, 16), jnp.float32),
        mesh=mesh,
        scratch_types=[
            pltpu.VMEM((2 * NE, 128), jnp.float32),
            pltpu.VMEM((2 * NE, 128), jnp.float32),
            pltpu.VMEM((3 * NA, 128), jnp.float32),
            pltpu.VMEM((16,), jnp.float32),
            pltpu.SemaphoreType.DMA,
            pltpu.SemaphoreType.DMA,
        ],
        compiler_params=pltpu.CompilerParams(needs_layout_passes=False),
    )
    return jnp.sum(f(mov4, loc4, goal4))


def kernel(movements, goal_predictions, utterances, locations, goals):
    mov4 = movements.transpose(1, 2, 0).reshape(NE, 2, NT, 128).transpose(0, 2, 1, 3)
    loc4 = locations.transpose(1, 2, 0).reshape(NE, 2, NT, 128).transpose(0, 2, 1, 3)
    goal4 = goals.transpose(2, 1, 0).reshape(3, NA, NT, 128)
    return _sc_cost(mov4, loc4, goal4)
